# baseline (speedup 1.0000x reference)
import dataclasses

import jax
import jax.numpy as jnp
from jax import lax
from jax.experimental import pallas as pl
from jax.experimental.pallas import tpu as pltpu
from jax.experimental.pallas import tpu_sc as plsc

N = 10000
E = 320000
D = 128
LAM = 0.5

NC = 2
NS = 16
NW = NC * NS
CHUNK = 80
EDGES_PER_TILE = E // NW
CHUNKS_PER_TILE = EDGES_PER_TILE // CHUNK
CHG = 4
NITER = CHUNKS_PER_TILE // CHG
NPAD = N
ZSTEP = 624
ZSPAN = 640

ROW_BLOCK = 1000


def _mlp_body(h_ref, w_ref, b_ref, z_ref):
    z = lax.dot_general(
        h_ref[...], w_ref[...], (((1,), (0,)), ((), ())),
        preferred_element_type=jnp.float32,
        precision=lax.Precision.HIGHEST,
    )
    z = jnp.maximum(z + b_ref[...], 0.0)
    nrm = jnp.sqrt(jnp.sum(z * z, axis=1, keepdims=True))
    z_ref[...] = z / jnp.maximum(nrm, 1e-12)


def _mlp(h, W, b2d):
    return pl.pallas_call(
        _mlp_body,
        grid=(N // ROW_BLOCK,),
        in_specs=[
            pl.BlockSpec((ROW_BLOCK, D), lambda i: (i, 0)),
            pl.BlockSpec((D, D), lambda i: (0, 0)),
            pl.BlockSpec((1, D), lambda i: (0, 0)),
        ],
        out_specs=pl.BlockSpec((ROW_BLOCK, D), lambda i: (i, 0)),
        out_shape=jax.ShapeDtypeStruct((N, D), jnp.float32),
    )(h, W, b2d)


def _make_hop(with_deg):
    mesh = plsc.VectorSubcoreMesh(core_axis_name="c", subcore_axis_name="s")

    out_type = [jax.ShapeDtypeStruct((NC, N, D), jnp.float32)]
    scratch = [
        pltpu.VMEM((2, CHUNK), jnp.int32),
        pltpu.VMEM((2, CHUNK), jnp.int32),
        pltpu.VMEM((2, CHUNK), jnp.int32),
        pltpu.VMEM((2, CHUNK), jnp.int32),
        pltpu.VMEM((CHUNK, D), jnp.float32),
        pltpu.VMEM((CHUNK, D), jnp.float32),
        pltpu.VMEM_SHARED((NPAD, D), jnp.float32),
    ]
    if with_deg:
        out_type.append(jax.ShapeDtypeStruct((NW, 8, NPAD), jnp.float32))
        scratch.append(pltpu.VMEM((NPAD,), jnp.float32))
    scratch += [pltpu.SemaphoreType.DMA] * 6

    def inner(table, sd3, zrows, out, degout, refs):
        (s0, s1, s2, s3, r0, r1, acc, degt,
         si0, si1, si2, si3, sg0, sg1) = refs
        slots = [s0, s1, s2, s3]
        sem_i = [si0, si1, si2, si3]
        rows = [r0, r1]
        sg = [sg0, sg1]

        c = lax.axis_index("c")
        s = lax.axis_index("s")
        w = c * NS + s
        row0 = pl.multiple_of(s * ZSTEP, 8)
        cbase = w * CHUNKS_PER_TILE
        pltpu.sync_copy(zrows, acc.at[pl.ds(row0, ZSPAN)])
        if with_deg:
            @pl.loop(0, NPAD // 16)
            def _(i):
                degt[pl.ds(pl.multiple_of(i * 16, 16), 16)] = jnp.zeros(
                    (16,), jnp.float32)
        plsc.subcore_barrier()

        def deg_update(slot):
            if with_deg:
                for t in range(CHUNK // 16):
                    iv = slot[1, pl.ds(t * 16, 16)]
                    plsc.addupdate_scatter(degt, [iv],
                                           jnp.ones((16,), jnp.float32))

        @pl.loop(0, NITER)
        def _(it):
            base = cbase + it * CHG
            hi = [pltpu.async_copy(sd3.at[base + j], slots[j], sem_i[j])
                  for j in range(CHG)]
            gs = [None] * CHG
            hi[0].wait()
            gs[0] = pltpu.async_copy(table.at[slots[0].at[0]], rows[0], sg[0])
            hi[1].wait()
            gs[1] = pltpu.async_copy(table.at[slots[1].at[0]], rows[1], sg[1])
            for j in range(CHG):
                gs[j].wait()
                pltpu.sync_copy(rows[j % 2], acc.at[slots[j].at[1]], add=True)
                if j + 2 < CHG:
                    hi[j + 2].wait()
                    gs[j + 2] = pltpu.async_copy(
                        table.at[slots[j + 2].at[0]], rows[j % 2], sg[j % 2])
                deg_update(slots[j])

        pltpu.sync_copy(sd3.at[cbase + NITER * CHG], slots[0])
        pltpu.sync_copy(table.at[slots[0].at[0]], rows[0])
        pltpu.sync_copy(rows[0], acc.at[slots[0].at[1]], add=True)
        deg_update(slots[0])

        plsc.subcore_barrier()
        pltpu.sync_copy(acc.at[pl.ds(row0, ZSPAN)],
                        out.at[c, pl.ds(row0, ZSPAN)])
        if with_deg:
            pltpu.sync_copy(degt, degout.at[w, 0])

    if with_deg:
        def body(table, sd3, zrows, out, degout, *refs):
            inner(table, sd3, zrows, out, degout, refs)
    else:
        def body(table, sd3, zrows, out, *refs):
            refs = refs[:7] + (None,) + refs[7:]
            inner(table, sd3, zrows, out, None, refs)

    cp = pltpu.CompilerParams()
    if "needs_layout_passes" in pltpu.CompilerParams.__dataclass_fields__:
        cp = dataclasses.replace(cp, needs_layout_passes=False)
    return pl.kernel(body, out_type=out_type, mesh=mesh,
                     scratch_types=scratch, compiler_params=cp)


_hop_deg = _make_hop(True)
_hop = _make_hop(False)


def _c1_body(p_ref, pd_ref, out_ref):
    s = p_ref[0] + p_ref[1]
    deg = jnp.sum(pd_ref[:, 0, :], axis=0)[:N]
    out_ref[...] = s / jnp.maximum(deg, 1.0)[:, None]


def _combine1(p, pdeg):
    return pl.pallas_call(
        _c1_body,
        grid=(1,),
        in_specs=[
            pl.BlockSpec((NC, N, D), lambda i: (0, 0, 0)),
            pl.BlockSpec((NW, 8, NPAD), lambda i: (0, 0, 0)),
        ],
        out_specs=pl.BlockSpec((N, D), lambda i: (0, 0)),
        out_shape=jax.ShapeDtypeStruct((N, D), jnp.float32),
    )(p, pdeg)


def _c2_body(n1_ref, p_ref, pd_ref, out_ref):
    s = p_ref[0] + p_ref[1]
    deg = jnp.sum(pd_ref[:, 0, :], axis=0)[:N]
    neigh2 = s / jnp.maximum(deg, 1.0)[:, None]
    out_ref[...] = LAM * n1_ref[...] + (1.0 - LAM) * neigh2


def _combine2(n1, p, pdeg):
    return pl.pallas_call(
        _c2_body,
        grid=(1,),
        in_specs=[
            pl.BlockSpec((N, D), lambda i: (0, 0)),
            pl.BlockSpec((NC, N, D), lambda i: (0, 0, 0)),
            pl.BlockSpec((NW, 8, NPAD), lambda i: (0, 0, 0)),
        ],
        out_specs=pl.BlockSpec((N, D), lambda i: (0, 0)),
        out_shape=jax.ShapeDtypeStruct((N, D), jnp.float32),
    )(n1, p, pdeg)


def kernel(h, edge_index, W, b):
    z = _mlp(h, W, b.reshape(1, D))
    sd3 = jnp.stack([edge_index[0].reshape(E // CHUNK, CHUNK),
                     edge_index[1].reshape(E // CHUNK, CHUNK)], axis=1)
    zrows = jnp.zeros((ZSPAN, D), jnp.float32)
    p1, pdeg = _hop_deg(z, sd3, zrows)
    neigh1 = _combine1(p1, pdeg)
    (p2,) = _hop(neigh1, sd3, zrows)
    result = _combine2(neigh1, p2, pdeg)
    return (z, result)


# speedup vs baseline: 1.0133x; 1.0133x over previous
import dataclasses

import jax
import jax.numpy as jnp
from jax import lax
from jax.experimental import pallas as pl
from jax.experimental.pallas import tpu as pltpu
from jax.experimental.pallas import tpu_sc as plsc

N = 10000
E = 320000
D = 128
LAM = 0.5

NC = 2
NS = 16
NW = NC * NS
CHUNK = 80
EDGES_PER_TILE = E // NW
CHUNKS_PER_TILE = EDGES_PER_TILE // CHUNK
CHG = 5
NITER = CHUNKS_PER_TILE // CHG
NPAD = N
ZSTEP = 624
ZSPAN = 640

ROW_BLOCK = 1000


def _mlp_body(h_ref, w_ref, b_ref, z_ref):
    z = lax.dot_general(
        h_ref[...], w_ref[...], (((1,), (0,)), ((), ())),
        preferred_element_type=jnp.float32,
        precision=lax.Precision.HIGHEST,
    )
    z = jnp.maximum(z + b_ref[...], 0.0)
    nrm = jnp.sqrt(jnp.sum(z * z, axis=1, keepdims=True))
    z_ref[...] = z / jnp.maximum(nrm, 1e-12)


def _mlp(h, W, b2d):
    return pl.pallas_call(
        _mlp_body,
        grid=(N // ROW_BLOCK,),
        in_specs=[
            pl.BlockSpec((ROW_BLOCK, D), lambda i: (i, 0)),
            pl.BlockSpec((D, D), lambda i: (0, 0)),
            pl.BlockSpec((1, D), lambda i: (0, 0)),
        ],
        out_specs=pl.BlockSpec((ROW_BLOCK, D), lambda i: (i, 0)),
        out_shape=jax.ShapeDtypeStruct((N, D), jnp.float32),
    )(h, W, b2d)


def _make_hop(with_deg):
    mesh = plsc.VectorSubcoreMesh(core_axis_name="c", subcore_axis_name="s")

    out_type = [jax.ShapeDtypeStruct((NC, N, D), jnp.float32)]
    scratch = [
        pltpu.VMEM((2, CHUNK), jnp.int32),
        pltpu.VMEM((2, CHUNK), jnp.int32),
        pltpu.VMEM((2, CHUNK), jnp.int32),
        pltpu.VMEM((2, CHUNK), jnp.int32),
        pltpu.VMEM((2, CHUNK), jnp.int32),
        pltpu.VMEM((CHUNK, D), jnp.float32),
        pltpu.VMEM((CHUNK, D), jnp.float32),
        pltpu.VMEM_SHARED((NPAD, D), jnp.float32),
    ]
    if with_deg:
        out_type.append(jax.ShapeDtypeStruct((NW, 8, NPAD), jnp.float32))
        scratch.append(pltpu.VMEM((NPAD,), jnp.float32))
    scratch += [pltpu.SemaphoreType.DMA] * 7

    def inner(table, sd3, zrows, out, degout, refs):
        (s0, s1, s2, s3, s4, r0, r1, acc, degt,
         si0, si1, si2, si3, si4, sg0, sg1) = refs
        slots = [s0, s1, s2, s3, s4]
        sem_i = [si0, si1, si2, si3, si4]
        rows = [r0, r1]
        sg = [sg0, sg1]

        c = lax.axis_index("c")
        s = lax.axis_index("s")
        w = c * NS + s
        row0 = pl.multiple_of(s * ZSTEP, 8)
        cbase = w * CHUNKS_PER_TILE
        pltpu.sync_copy(zrows, acc.at[pl.ds(row0, ZSPAN)])
        if with_deg:
            @pl.loop(0, NPAD // 16)
            def _(i):
                degt[pl.ds(pl.multiple_of(i * 16, 16), 16)] = jnp.zeros(
                    (16,), jnp.float32)
        plsc.subcore_barrier()

        def deg_update(slot):
            if with_deg:
                for t in range(CHUNK // 16):
                    iv = slot[1, pl.ds(t * 16, 16)]
                    plsc.addupdate_scatter(degt, [iv],
                                           jnp.ones((16,), jnp.float32))

        @pl.loop(0, NITER)
        def _(it):
            base = cbase + it * CHG
            hi = [pltpu.async_copy(sd3.at[base + j], slots[j], sem_i[j])
                  for j in range(CHG)]
            gs = [None] * CHG
            hi[0].wait()
            gs[0] = pltpu.async_copy(table.at[slots[0].at[0]], rows[0], sg[0])
            hi[1].wait()
            gs[1] = pltpu.async_copy(table.at[slots[1].at[0]], rows[1], sg[1])
            for j in range(CHG):
                gs[j].wait()
                pltpu.sync_copy(rows[j % 2], acc.at[slots[j].at[1]], add=True)
                if j + 2 < CHG:
                    hi[j + 2].wait()
                    gs[j + 2] = pltpu.async_copy(
                        table.at[slots[j + 2].at[0]], rows[j % 2], sg[j % 2])
                deg_update(slots[j])

        plsc.subcore_barrier()
        pltpu.sync_copy(acc.at[pl.ds(row0, ZSPAN)],
                        out.at[c, pl.ds(row0, ZSPAN)])
        if with_deg:
            pltpu.sync_copy(degt, degout.at[w, 0])

    if with_deg:
        def body(table, sd3, zrows, out, degout, *refs):
            inner(table, sd3, zrows, out, degout, refs)
    else:
        def body(table, sd3, zrows, out, *refs):
            refs = refs[:8] + (None,) + refs[8:]
            inner(table, sd3, zrows, out, None, refs)

    cp = pltpu.CompilerParams()
    if "needs_layout_passes" in pltpu.CompilerParams.__dataclass_fields__:
        cp = dataclasses.replace(cp, needs_layout_passes=False)
    return pl.kernel(body, out_type=out_type, mesh=mesh,
                     scratch_types=scratch, compiler_params=cp)


_hop_deg = _make_hop(True)
_hop = _make_hop(False)


def _c1_body(p_ref, pd_ref, out_ref):
    s = p_ref[0] + p_ref[1]
    deg = jnp.sum(pd_ref[:, 0, :], axis=0)[:N]
    out_ref[...] = s / jnp.maximum(deg, 1.0)[:, None]


def _combine1(p, pdeg):
    return pl.pallas_call(
        _c1_body,
        grid=(1,),
        in_specs=[
            pl.BlockSpec((NC, N, D), lambda i: (0, 0, 0)),
            pl.BlockSpec((NW, 8, NPAD), lambda i: (0, 0, 0)),
        ],
        out_specs=pl.BlockSpec((N, D), lambda i: (0, 0)),
        out_shape=jax.ShapeDtypeStruct((N, D), jnp.float32),
    )(p, pdeg)


def _c2_body(n1_ref, p_ref, pd_ref, out_ref):
    s = p_ref[0] + p_ref[1]
    deg = jnp.sum(pd_ref[:, 0, :], axis=0)[:N]
    neigh2 = s / jnp.maximum(deg, 1.0)[:, None]
    out_ref[...] = LAM * n1_ref[...] + (1.0 - LAM) * neigh2


def _combine2(n1, p, pdeg):
    return pl.pallas_call(
        _c2_body,
        grid=(1,),
        in_specs=[
            pl.BlockSpec((N, D), lambda i: (0, 0)),
            pl.BlockSpec((NC, N, D), lambda i: (0, 0, 0)),
            pl.BlockSpec((NW, 8, NPAD), lambda i: (0, 0, 0)),
        ],
        out_specs=pl.BlockSpec((N, D), lambda i: (0, 0)),
        out_shape=jax.ShapeDtypeStruct((N, D), jnp.float32),
    )(n1, p, pdeg)


def kernel(h, edge_index, W, b):
    z = _mlp(h, W, b.reshape(1, D))
    sd3 = jnp.stack([edge_index[0].reshape(E // CHUNK, CHUNK),
                     edge_index[1].reshape(E // CHUNK, CHUNK)], axis=1)
    zrows = jnp.zeros((ZSPAN, D), jnp.float32)
    p1, pdeg = _hop_deg(z, sd3, zrows)
    neigh1 = _combine1(p1, pdeg)
    (p2,) = _hop(neigh1, sd3, zrows)
    result = _combine2(neigh1, p2, pdeg)
    return (z, result)


# speedup vs baseline: 1.0752x; 1.0611x over previous
import dataclasses

import jax
import jax.numpy as jnp
from jax import lax
from jax.experimental import pallas as pl
from jax.experimental.pallas import tpu as pltpu
from jax.experimental.pallas import tpu_sc as plsc

N = 10000
E = 320000
D = 128
LAM = 0.5

NC = 2
NS = 16
NW = NC * NS
CHUNK = 80
EDGES_PER_TILE = E // NW
CHUNKS_PER_TILE = EDGES_PER_TILE // CHUNK
CHG = 5
NITER = CHUNKS_PER_TILE // CHG
NPAD = N
ZSTEP = 624
ZSPAN = 640

ROW_BLOCK = 1000


def _mlp_body(h_ref, w_ref, b_ref, z_ref):
    z = lax.dot_general(
        h_ref[...], w_ref[...], (((1,), (0,)), ((), ())),
        preferred_element_type=jnp.float32,
        precision=lax.Precision.HIGHEST,
    )
    z = jnp.maximum(z + b_ref[...], 0.0)
    nrm = jnp.sqrt(jnp.sum(z * z, axis=1, keepdims=True))
    z_ref[...] = z / jnp.maximum(nrm, 1e-12)


def _mlp(h, W, b2d):
    return pl.pallas_call(
        _mlp_body,
        grid=(N // ROW_BLOCK,),
        in_specs=[
            pl.BlockSpec((ROW_BLOCK, D), lambda i: (i, 0)),
            pl.BlockSpec((D, D), lambda i: (0, 0)),
            pl.BlockSpec((1, D), lambda i: (0, 0)),
        ],
        out_specs=pl.BlockSpec((ROW_BLOCK, D), lambda i: (i, 0)),
        out_shape=jax.ShapeDtypeStruct((N, D), jnp.float32),
    )(h, W, b2d)


def _make_hop(with_deg):
    mesh = plsc.VectorSubcoreMesh(core_axis_name="c", subcore_axis_name="s")

    out_type = [jax.ShapeDtypeStruct((NC, N, D), jnp.float32)]
    scratch = [
        pltpu.VMEM((2, CHUNK), jnp.int32),
        pltpu.VMEM((2, CHUNK), jnp.int32),
        pltpu.VMEM((2, CHUNK), jnp.int32),
        pltpu.VMEM((2, CHUNK), jnp.int32),
        pltpu.VMEM((2, CHUNK), jnp.int32),
        pltpu.VMEM((CHUNK, D), jnp.float32),
        pltpu.VMEM((CHUNK, D), jnp.float32),
        pltpu.VMEM((CHUNK, D), jnp.float32),
        pltpu.VMEM_SHARED((NPAD, D), jnp.float32),
    ]
    if with_deg:
        out_type.append(jax.ShapeDtypeStruct((NW, 8, NPAD), jnp.float32))
        scratch.append(pltpu.VMEM((NPAD,), jnp.float32))
    scratch += [pltpu.SemaphoreType.DMA] * 11

    def inner(table, sd3, zrows, out, degout, refs):
        (s0, s1, s2, s3, s4, r0, r1, r2, acc, degt,
         si0, si1, si2, si3, si4, sg0, sg1, sg2, ss0, ss1, ss2) = refs
        slots = [s0, s1, s2, s3, s4]
        sem_i = [si0, si1, si2, si3, si4]
        rows = [r0, r1, r2]
        sg = [sg0, sg1, sg2]
        ss = [ss0, ss1, ss2]

        c = lax.axis_index("c")
        s = lax.axis_index("s")
        w = c * NS + s
        row0 = pl.multiple_of(s * ZSTEP, 8)
        cbase = w * CHUNKS_PER_TILE
        pltpu.sync_copy(zrows, acc.at[pl.ds(row0, ZSPAN)])
        if with_deg:
            @pl.loop(0, NPAD // 16)
            def _(i):
                degt[pl.ds(pl.multiple_of(i * 16, 16), 16)] = jnp.zeros(
                    (16,), jnp.float32)
        plsc.subcore_barrier()

        def deg_update(slot):
            if with_deg:
                for t in range(CHUNK // 16):
                    iv = slot[1, pl.ds(t * 16, 16)]
                    plsc.addupdate_scatter(degt, [iv],
                                           jnp.ones((16,), jnp.float32))

        @pl.loop(0, NITER)
        def _(it):
            base = cbase + it * CHG
            hi = [pltpu.async_copy(sd3.at[base + j], slots[j], sem_i[j])
                  for j in range(CHG)]
            gs = [None] * CHG
            sc = [None] * CHG
            for j in range(3):
                hi[j].wait()
                gs[j] = pltpu.async_copy(table.at[slots[j].at[0]],
                                         rows[j], sg[j])
            for j in range(CHG):
                gs[j].wait()
                sc[j] = pltpu.async_copy(rows[j % 3], acc.at[slots[j].at[1]],
                                         ss[j % 3], add=True)
                deg_update(slots[j])
                if j + 3 < CHG:
                    sc[j].wait()
                    hi[j + 3].wait()
                    gs[j + 3] = pltpu.async_copy(
                        table.at[slots[j + 3].at[0]], rows[j % 3], sg[j % 3])
            for j in range(CHG - 3, CHG):
                sc[j].wait()

        plsc.subcore_barrier()
        pltpu.sync_copy(acc.at[pl.ds(row0, ZSPAN)],
                        out.at[c, pl.ds(row0, ZSPAN)])
        if with_deg:
            pltpu.sync_copy(degt, degout.at[w, 0])

    if with_deg:
        def body(table, sd3, zrows, out, degout, *refs):
            inner(table, sd3, zrows, out, degout, refs)
    else:
        def body(table, sd3, zrows, out, *refs):
            refs = refs[:9] + (None,) + refs[9:]
            inner(table, sd3, zrows, out, None, refs)

    cp = pltpu.CompilerParams()
    if "needs_layout_passes" in pltpu.CompilerParams.__dataclass_fields__:
        cp = dataclasses.replace(cp, needs_layout_passes=False)
    return pl.kernel(body, out_type=out_type, mesh=mesh,
                     scratch_types=scratch, compiler_params=cp)


_hop_deg = _make_hop(True)
_hop = _make_hop(False)


def _c1_body(p_ref, pd_ref, out_ref):
    s = p_ref[0] + p_ref[1]
    deg = jnp.sum(pd_ref[:, 0, :], axis=0)[:N]
    out_ref[...] = s / jnp.maximum(deg, 1.0)[:, None]


def _combine1(p, pdeg):
    return pl.pallas_call(
        _c1_body,
        grid=(1,),
        in_specs=[
            pl.BlockSpec((NC, N, D), lambda i: (0, 0, 0)),
            pl.BlockSpec((NW, 8, NPAD), lambda i: (0, 0, 0)),
        ],
        out_specs=pl.BlockSpec((N, D), lambda i: (0, 0)),
        out_shape=jax.ShapeDtypeStruct((N, D), jnp.float32),
    )(p, pdeg)


def _c2_body(n1_ref, p_ref, pd_ref, out_ref):
    s = p_ref[0] + p_ref[1]
    deg = jnp.sum(pd_ref[:, 0, :], axis=0)[:N]
    neigh2 = s / jnp.maximum(deg, 1.0)[:, None]
    out_ref[...] = LAM * n1_ref[...] + (1.0 - LAM) * neigh2


def _combine2(n1, p, pdeg):
    return pl.pallas_call(
        _c2_body,
        grid=(1,),
        in_specs=[
            pl.BlockSpec((N, D), lambda i: (0, 0)),
            pl.BlockSpec((NC, N, D), lambda i: (0, 0, 0)),
            pl.BlockSpec((NW, 8, NPAD), lambda i: (0, 0, 0)),
        ],
        out_specs=pl.BlockSpec((N, D), lambda i: (0, 0)),
        out_shape=jax.ShapeDtypeStruct((N, D), jnp.float32),
    )(n1, p, pdeg)


def kernel(h, edge_index, W, b):
    z = _mlp(h, W, b.reshape(1, D))
    sd3 = jnp.stack([edge_index[0].reshape(E // CHUNK, CHUNK),
                     edge_index[1].reshape(E // CHUNK, CHUNK)], axis=1)
    zrows = jnp.zeros((ZSPAN, D), jnp.float32)
    p1, pdeg = _hop_deg(z, sd3, zrows)
    neigh1 = _combine1(p1, pdeg)
    (p2,) = _hop(neigh1, sd3, zrows)
    result = _combine2(neigh1, p2, pdeg)
    return (z, result)
